# baseline (speedup 1.0000x reference)
import jax
import jax.numpy as jnp
from jax import lax
from jax.experimental import pallas as pl
from jax.experimental.pallas import tpu as pltpu
from jax.experimental.pallas import tpu_sc as plsc

S = 64
K = 256
UNROLL = 32


def _spmv_body(x_hbm, idx_hbm, vals_hbm, out_hbm,
               x_s, idx_s, vals_s, acc_a, acc_b, acc_c, acc_d, out_s, sem):
    cp_x = pltpu.make_async_copy(x_hbm, x_s, sem)
    cp_i = pltpu.make_async_copy(idx_hbm, idx_s, sem)
    cp_v = pltpu.make_async_copy(vals_hbm, vals_s, sem)
    cp_x.start()
    cp_i.start()
    cp_v.start()

    accs = (acc_a, acc_b, acc_c, acc_d)

    def zstep(j, carry):
        acc_a[j] = 0.0
        acc_b[j] = 0.0
        acc_c[j] = 0.0
        acc_d[j] = 0.0
        return carry

    lax.fori_loop(0, S, zstep, 0)

    cp_x.wait()
    cp_i.wait()
    cp_v.wait()

    def step(b, carry):
        base = b * UNROLL
        for u in range(UNROLL):
            k = base + u
            r = idx_s[0, k]
            c = idx_s[1, k]
            acc = accs[u % 4]
            acc[r] = acc[r] + vals_s[k] * x_s[c]
        return carry

    lax.fori_loop(0, K // UNROLL, step, 0)

    def cstep(j, carry):
        out_s[j] = (acc_a[j] + acc_b[j]) + (acc_c[j] + acc_d[j])
        return carry

    lax.fori_loop(0, S, cstep, 0)

    pltpu.sync_copy(out_s, out_hbm)


@jax.jit
def _spmv(x, idx, vals):
    mesh = plsc.ScalarSubcoreMesh(axis_name="c", num_cores=1)
    return pl.kernel(
        _spmv_body,
        out_type=jax.ShapeDtypeStruct((S,), jnp.float32),
        mesh=mesh,
        scratch_types=[
            pltpu.SMEM((S,), jnp.float32),
            pltpu.SMEM((2, K), jnp.int32),
            pltpu.SMEM((K,), jnp.float32),
            pltpu.SMEM((S,), jnp.float32),
            pltpu.SMEM((S,), jnp.float32),
            pltpu.SMEM((S,), jnp.float32),
            pltpu.SMEM((S,), jnp.float32),
            pltpu.SMEM((S,), jnp.float32),
            pltpu.SemaphoreType.DMA,
        ],
        compiler_params=pltpu.CompilerParams(needs_layout_passes=False),
    )(x, idx, vals)


def kernel(x, indices, values):
    return _spmv(x, indices.astype(jnp.int32), values)


# speedup vs baseline: 1.0577x; 1.0577x over previous
import jax
import jax.numpy as jnp
from jax import lax
from jax.experimental import pallas as pl
from jax.experimental.pallas import tpu as pltpu
from jax.experimental.pallas import tpu_sc as plsc

S = 64
K = 256
L = 16


def _spmv_body(x_hbm, idx_hbm, vals_hbm, out_hbm,
               x_v, idx_v, vals_v, acc_a, acc_b, sem):
    cp_x = pltpu.make_async_copy(x_hbm, x_v, sem)
    cp_i = pltpu.make_async_copy(idx_hbm, idx_v, sem)
    cp_v = pltpu.make_async_copy(vals_hbm, vals_v, sem)
    cp_x.start()
    cp_i.start()
    cp_v.start()

    zero = jnp.zeros((L,), jnp.float32)
    for j in range(S // L):
        acc_a[pl.ds(j * L, L)] = zero
        acc_b[pl.ds(j * L, L)] = zero

    cp_x.wait()
    cp_i.wait()
    cp_v.wait()

    for i in range(K // L):
        r = idx_v[0, pl.ds(i * L, L)]
        c = idx_v[1, pl.ds(i * L, L)]
        v = vals_v[pl.ds(i * L, L)]
        g = plsc.load_gather(x_v, [c])
        acc = acc_a if i % 2 == 0 else acc_b
        plsc.addupdate_scatter(acc, [r], v * g)

    for j in range(S // L):
        sl = pl.ds(j * L, L)
        acc_a[sl] = acc_a[sl] + acc_b[sl]

    pltpu.sync_copy(acc_a, out_hbm)


@jax.jit
def _spmv(x, idx, vals):
    mesh = plsc.VectorSubcoreMesh(
        core_axis_name="c", subcore_axis_name="s",
        num_cores=1, num_subcores=1)
    return pl.kernel(
        _spmv_body,
        out_type=jax.ShapeDtypeStruct((S,), jnp.float32),
        mesh=mesh,
        scratch_types=[
            pltpu.VMEM((S,), jnp.float32),
            pltpu.VMEM((2, K), jnp.int32),
            pltpu.VMEM((K,), jnp.float32),
            pltpu.VMEM((S,), jnp.float32),
            pltpu.VMEM((S,), jnp.float32),
            pltpu.SemaphoreType.DMA,
        ],
        compiler_params=pltpu.CompilerParams(needs_layout_passes=False),
    )(x, idx, vals)


def kernel(x, indices, values):
    return _spmv(x, indices.astype(jnp.int32), values)


# speedup vs baseline: 1.0654x; 1.0073x over previous
import jax
import jax.numpy as jnp
from jax import lax
from jax.experimental import pallas as pl
from jax.experimental.pallas import tpu as pltpu
from jax.experimental.pallas import tpu_sc as plsc

S = 64
K = 256
L = 16


def _spmv_body(x_hbm, idx_hbm, vals_hbm, out_hbm,
               x_v, idx_v, vals_v, acc_a, acc_b, acc_c, acc_d, sem):
    cp_x = pltpu.make_async_copy(x_hbm, x_v, sem)
    cp_i = pltpu.make_async_copy(idx_hbm, idx_v, sem)
    cp_v = pltpu.make_async_copy(vals_hbm, vals_v, sem)
    cp_x.start()
    cp_i.start()
    cp_v.start()

    accs = (acc_a, acc_b, acc_c, acc_d)
    zero = jnp.zeros((L,), jnp.float32)
    for j in range(S // L):
        for acc in accs:
            acc[pl.ds(j * L, L)] = zero

    cp_x.wait()
    cp_i.wait()
    cp_v.wait()

    rs = []
    gs = []
    for i in range(K // L):
        r = idx_v[0, pl.ds(i * L, L)]
        c = idx_v[1, pl.ds(i * L, L)]
        v = vals_v[pl.ds(i * L, L)]
        rs.append(r)
        gs.append(plsc.load_gather(x_v, [c]) * v)

    for i in range(K // L):
        plsc.addupdate_scatter(accs[i % 4], [rs[i]], gs[i])

    for j in range(S // L):
        sl = pl.ds(j * L, L)
        acc_a[sl] = (acc_a[sl] + acc_b[sl]) + (acc_c[sl] + acc_d[sl])

    pltpu.sync_copy(acc_a, out_hbm)


@jax.jit
def _spmv(x, idx, vals):
    mesh = plsc.VectorSubcoreMesh(
        core_axis_name="c", subcore_axis_name="s",
        num_cores=1, num_subcores=1)
    return pl.kernel(
        _spmv_body,
        out_type=jax.ShapeDtypeStruct((S,), jnp.float32),
        mesh=mesh,
        scratch_types=[
            pltpu.VMEM((S,), jnp.float32),
            pltpu.VMEM((2, K), jnp.int32),
            pltpu.VMEM((K,), jnp.float32),
            pltpu.VMEM((S,), jnp.float32),
            pltpu.VMEM((S,), jnp.float32),
            pltpu.VMEM((S,), jnp.float32),
            pltpu.VMEM((S,), jnp.float32),
            pltpu.SemaphoreType.DMA,
        ],
        compiler_params=pltpu.CompilerParams(needs_layout_passes=False),
    )(x, idx, vals)


def kernel(x, indices, values):
    return _spmv(x, indices.astype(jnp.int32), values)


# speedup vs baseline: 1.0681x; 1.0025x over previous
import jax
import jax.numpy as jnp
from jax import lax
from jax.experimental import pallas as pl
from jax.experimental.pallas import tpu as pltpu
from jax.experimental.pallas import tpu_sc as plsc

S = 64
K = 256
L = 16


def _spmv_body(x_hbm, idx_hbm, vals_hbm, out_hbm,
               x_v, idx_v, vals_v, acc_a, acc_b, acc_c, acc_d, sem):
    cp_x = pltpu.make_async_copy(x_hbm, x_v, sem)
    cp_i = pltpu.make_async_copy(idx_hbm, idx_v, sem)
    cp_v = pltpu.make_async_copy(vals_hbm, vals_v, sem)
    cp_x.start()
    cp_i.start()
    cp_v.start()

    accs = (acc_a, acc_b, acc_c, acc_d)
    zero = jnp.zeros((L,), jnp.float32)
    for j in range(S // L):
        for acc in accs:
            acc[pl.ds(j * L, L)] = zero

    cp_x.wait()
    cp_i.wait()
    cp_v.wait()

    rs = []
    gs = []
    for i in range(K // L):
        r = idx_v[0, pl.ds(i * L, L)]
        c = idx_v[1, pl.ds(i * L, L)]
        v = vals_v[pl.ds(i * L, L)]
        rs.append(r)
        gs.append(plsc.load_gather(x_v, [c]) * v)

    for i in range(K // L):
        plsc.addupdate_scatter(accs[i % 4], [rs[i]], gs[i])

    for j in range(S // L):
        sl = pl.ds(j * L, L)
        acc_a[sl] = (acc_a[sl] + acc_b[sl]) + (acc_c[sl] + acc_d[sl])

    pltpu.sync_copy(acc_a, out_hbm)


@jax.jit
def _spmv(x, idx, vals):
    mesh = plsc.VectorSubcoreMesh(
        core_axis_name="c", subcore_axis_name="s",
        num_cores=1, num_subcores=1)
    return pl.kernel(
        _spmv_body,
        out_type=jax.ShapeDtypeStruct((S,), jnp.float32),
        mesh=mesh,
        scratch_types=[
            pltpu.VMEM((S,), jnp.float32),
            pltpu.VMEM((2, K), jnp.int32),
            pltpu.VMEM((K,), jnp.float32),
            pltpu.VMEM((S,), jnp.float32),
            pltpu.VMEM((S,), jnp.float32),
            pltpu.VMEM((S,), jnp.float32),
            pltpu.VMEM((S,), jnp.float32),
            pltpu.SemaphoreType.DMA,
        ],
        compiler_params=pltpu.CompilerParams(
            needs_layout_passes=False, skip_device_barrier=True),
    )(x, idx, vals)


def kernel(x, indices, values):
    return _spmv(x, indices.astype(jnp.int32), values)


# speedup vs baseline: 1.0876x; 1.0182x over previous
import jax
import jax.numpy as jnp
from jax import lax
from jax.experimental import pallas as pl
from jax.experimental.pallas import tpu as pltpu
from jax.experimental.pallas import tpu_sc as plsc
from jax._src.pallas import mpmd

S = 64
K = 256
L = 16

_SMESH = plsc.ScalarSubcoreMesh(axis_name="c", num_cores=1)
_VMESH = plsc.VectorSubcoreMesh(
    core_axis_name="c", subcore_axis_name="s", num_cores=1, num_subcores=1)


def _scs_fn(x_hbm, idx_hbm, vals_hbm, out_hbm,
            x_sp, idx_sp, vals_sp, scs_sem, rdy):
    cp_x = pltpu.make_async_copy(x_hbm, x_sp, scs_sem)
    cp_i = pltpu.make_async_copy(idx_hbm, idx_sp, scs_sem)
    cp_v = pltpu.make_async_copy(vals_hbm, vals_sp, scs_sem)
    cp_x.start()
    cp_i.start()
    cp_v.start()
    cp_x.wait()
    cp_i.wait()
    cp_v.wait()
    pl.semaphore_signal(
        rdy, 1, device_id={"c": 0, "s": 0},
        device_id_type=pl.DeviceIdType.MESH)


def _tec_fn(x_hbm, idx_hbm, vals_hbm, out_hbm,
            x_sp, idx_sp, vals_sp, scs_sem, rdy):
    def scoped(x_v, idx_v, vals_v, acc_a, acc_b, acc_c, acc_d, tec_sem):
        accs = (acc_a, acc_b, acc_c, acc_d)
        zero = jnp.zeros((L,), jnp.float32)
        for j in range(S // L):
            for acc in accs:
                acc[pl.ds(j * L, L)] = zero

        pl.semaphore_wait(rdy, 1)

        cp_x = pltpu.make_async_copy(x_sp, x_v, tec_sem)
        cp_i = pltpu.make_async_copy(idx_sp, idx_v, tec_sem)
        cp_v = pltpu.make_async_copy(vals_sp, vals_v, tec_sem)
        cp_x.start()
        cp_i.start()
        cp_v.start()
        cp_x.wait()
        cp_i.wait()
        cp_v.wait()

        rs = []
        gs = []
        for i in range(K // L):
            r = idx_v[0, pl.ds(i * L, L)]
            c = idx_v[1, pl.ds(i * L, L)]
            v = vals_v[pl.ds(i * L, L)]
            rs.append(r)
            gs.append(plsc.load_gather(x_v, [c]) * v)

        for i in range(K // L):
            plsc.addupdate_scatter(accs[i % 4], [rs[i]], gs[i])

        for j in range(S // L):
            sl = pl.ds(j * L, L)
            acc_a[sl] = (acc_a[sl] + acc_b[sl]) + (acc_c[sl] + acc_d[sl])

        pltpu.sync_copy(acc_a, out_hbm)

    pl.run_scoped(
        scoped,
        pltpu.VMEM((S,), jnp.float32),
        pltpu.VMEM((2, K), jnp.int32),
        pltpu.VMEM((K,), jnp.float32),
        pltpu.VMEM((S,), jnp.float32),
        pltpu.VMEM((S,), jnp.float32),
        pltpu.VMEM((S,), jnp.float32),
        pltpu.VMEM((S,), jnp.float32),
        pltpu.SemaphoreType.DMA,
    )


@jax.jit
def _spmv(x, idx, vals):
    return mpmd.mpmd_map(
        [(_SMESH, _scs_fn), (_VMESH, _tec_fn)],
        out_types=[jax.ShapeDtypeStruct((S,), jnp.float32)],
        scratch_types=[
            pltpu.VMEM_SHARED((S,), jnp.float32),
            pltpu.VMEM_SHARED((2, K), jnp.int32),
            pltpu.VMEM_SHARED((K,), jnp.float32),
            pltpu.SemaphoreType.DMA @ _SMESH,
            pltpu.SemaphoreType.REGULAR @ _VMESH,
        ],
        compiler_params=pltpu.CompilerParams(needs_layout_passes=False),
    )(x, idx, vals)[0]


def kernel(x, indices, values):
    return _spmv(x, indices.astype(jnp.int32), values)
